# baseline (speedup 1.0000x reference)
import jax
import jax.numpy as jnp
from jax.experimental import pallas as pl
from jax.experimental.pallas import tpu as pltpu

DEPTH = 1000
ROWS_PER_BLOCK = 512


def _one_hot_block(idx_ref, out_ref):
    idx = idx_ref[:].astype(jnp.int32)
    iota = jax.lax.broadcasted_iota(jnp.int32, (ROWS_PER_BLOCK, DEPTH), 1)
    out_ref[0] = (iota == idx).astype(jnp.float32)


def kernel(x):
    n = x.size
    x_col = jnp.reshape(x, (n, 1))
    num_blocks = n // ROWS_PER_BLOCK
    out = pl.pallas_call(
        _one_hot_block,
        grid=(num_blocks,),
        in_specs=[pl.BlockSpec((ROWS_PER_BLOCK, 1), lambda i: (i, 0))],
        out_specs=pl.BlockSpec((1, ROWS_PER_BLOCK, DEPTH), lambda i: (0, i, 0)),
        out_shape=jax.ShapeDtypeStruct((1, n, DEPTH), jnp.float32),
    )(x_col)
    return out


# speedup vs baseline: 1.1148x; 1.1148x over previous
import jax
import jax.numpy as jnp
from jax.experimental import pallas as pl
from jax.experimental.pallas import tpu as pltpu

DEPTH = 1000
ROWS_PER_BLOCK = 2048


def _one_hot_block(idx_ref, out_ref):
    idx = idx_ref[:].astype(jnp.int32)
    iota = jax.lax.broadcasted_iota(jnp.int32, (ROWS_PER_BLOCK, DEPTH), 1)
    out_ref[0] = (iota == idx).astype(jnp.float32)


def kernel(x):
    n = x.size
    x_col = jnp.reshape(x, (n, 1))
    num_blocks = n // ROWS_PER_BLOCK
    out = pl.pallas_call(
        _one_hot_block,
        grid=(num_blocks,),
        in_specs=[pl.BlockSpec((ROWS_PER_BLOCK, 1), lambda i: (i, 0))],
        out_specs=pl.BlockSpec((1, ROWS_PER_BLOCK, DEPTH), lambda i: (0, i, 0)),
        out_shape=jax.ShapeDtypeStruct((1, n, DEPTH), jnp.float32),
    )(x_col)
    return out


# speedup vs baseline: 4.5464x; 4.0782x over previous
import jax
import jax.numpy as jnp
from jax.experimental import pallas as pl

DEPTH = 1000
DEPTH_PER_BLOCK = 40


def _one_hot_t_block(idx_ref, out_ref):
    d0 = pl.program_id(0) * DEPTH_PER_BLOCK
    idx = idx_ref[:].astype(jnp.int32)
    n = idx_ref.shape[1]
    drow = jax.lax.broadcasted_iota(jnp.int32, (DEPTH_PER_BLOCK, n), 0) + d0
    out_ref[:] = (drow == idx).astype(jnp.float32)


def kernel(x):
    n = x.size
    x_row = jnp.reshape(x, (1, n))
    num_blocks = DEPTH // DEPTH_PER_BLOCK
    out_t = pl.pallas_call(
        _one_hot_t_block,
        grid=(num_blocks,),
        in_specs=[pl.BlockSpec((1, n), lambda i: (0, 0))],
        out_specs=pl.BlockSpec((DEPTH_PER_BLOCK, n), lambda i: (i, 0)),
        out_shape=jax.ShapeDtypeStruct((DEPTH, n), jnp.float32),
    )(x_row)
    return jnp.reshape(jnp.transpose(out_t), (1, n, DEPTH))
